# Initial kernel scaffold; baseline (speedup 1.0000x reference)
import jax
import jax.numpy as jnp
from jax.experimental import pallas as pl


def kernel(x_n, edge_index_n, edge_attr_n, x_n_batch, W_node, b_node, W_edge, b_edge, W_msg, b_msg, W_upd, b_upd):
    raise NotImplementedError("write your pallas kernel here")


# speedup vs baseline: 1.8206x; 1.8206x over previous
import functools

import jax
import jax.numpy as jnp
from jax import lax
from jax.experimental import pallas as pl
from jax.experimental.pallas import tpu as pltpu
from jax.experimental.pallas import tpu_sc as plsc

N_NODES = 10000
N_EDGES = 320000
F_N = 66
F_E = 12
H = 128
L = 4
B = 64

NC = 2
NS = 16
NW = NC * NS
EPW = N_EDGES // NW
CHUNK = 80
NCHUNK = EPW // CHUNK
RPT = 624
REM = N_NODES - NS * RPT
LG = H // 16

BLK_N = 1000
BLK_E = 2000


def _node_enc_body(x_ref, Wn_ref, bn_ref, Wt_ref, h_ref, hw_ref):
    h = jnp.maximum(
        jnp.dot(x_ref[...], Wn_ref[...], preferred_element_type=jnp.float32)
        + bn_ref[...][None, :], 0.0)
    h_ref[...] = h
    hw_ref[...] = jnp.dot(h, Wt_ref[...], preferred_element_type=jnp.float32)


def _node_encode(x_n, W_node, b_node, Wt0):
    return pl.pallas_call(
        _node_enc_body,
        grid=(N_NODES // BLK_N,),
        in_specs=[
            pl.BlockSpec((BLK_N, F_N), lambda i: (i, 0)),
            pl.BlockSpec((F_N, H), lambda i: (0, 0)),
            pl.BlockSpec((H,), lambda i: (0,)),
            pl.BlockSpec((H, H), lambda i: (0, 0)),
        ],
        out_specs=[
            pl.BlockSpec((BLK_N, H), lambda i: (i, 0)),
            pl.BlockSpec((BLK_N, H), lambda i: (i, 0)),
        ],
        out_shape=[jax.ShapeDtypeStruct((N_NODES, H), jnp.float32)] * 2,
    )(x_n, W_node, b_node, Wt0)


def _edge_pre_body(ea_ref, We_ref, be_ref, Wb_ref, bm_ref, o0, o1, o2, o3):
    e = jnp.maximum(
        jnp.dot(ea_ref[...], We_ref[...], preferred_element_type=jnp.float32)
        + be_ref[...][None, :], 0.0)
    outs = (o0, o1, o2, o3)
    for l in range(L):
        outs[l][...] = (
            jnp.dot(e, Wb_ref[l], preferred_element_type=jnp.float32)
            + bm_ref[l][None, :])


def _edge_precompute(edge_attr, W_edge, b_edge, Wm_bot, b_msg):
    return pl.pallas_call(
        _edge_pre_body,
        grid=(N_EDGES // BLK_E,),
        in_specs=[
            pl.BlockSpec((BLK_E, F_E), lambda i: (i, 0)),
            pl.BlockSpec((F_E, H), lambda i: (0, 0)),
            pl.BlockSpec((H,), lambda i: (0,)),
            pl.BlockSpec((L, H, H), lambda i: (0, 0, 0)),
            pl.BlockSpec((L, H), lambda i: (0, 0)),
        ],
        out_specs=[pl.BlockSpec((BLK_E, H), lambda i: (i, 0))] * L,
        out_shape=[jax.ShapeDtypeStruct((N_EDGES, H), jnp.float32)] * L,
    )(edge_attr, W_edge, b_edge, Wm_bot, b_msg)


def _edge_pass_body(hw_hbm, ew_hbm, src_hbm, dst_hbm, zeros_hbm, out_hbm,
                    src_v, dst_v, rows_v, ew_v, zbuf_v, acc_sh, sem):
    cid = lax.axis_index("c")
    sid = lax.axis_index("s")
    wid = sid * NC + cid
    ebase = wid * EPW
    rbase = sid * RPT

    pltpu.sync_copy(zeros_hbm, zbuf_v)

    def zrow(i, carry):
        r = pl.multiple_of(rbase + i * REM, 8)
        pltpu.sync_copy(zbuf_v, acc_sh.at[pl.ds(r, REM)])
        return carry

    lax.fori_loop(0, RPT // REM, zrow, 0)

    @pl.when(sid == 0)
    def _zero_tail():
        pltpu.sync_copy(zbuf_v, acc_sh.at[pl.ds(NS * RPT, REM)])

    plsc.subcore_barrier()

    def chunk(ci, carry):
        off = ebase + ci * CHUNK
        pltpu.sync_copy(src_hbm.at[pl.ds(off, CHUNK)], src_v)
        pltpu.sync_copy(dst_hbm.at[pl.ds(off, CHUNK)], dst_v)
        pltpu.sync_copy(ew_hbm.at[pl.ds(off, CHUNK)], ew_v)
        pltpu.async_copy(hw_hbm.at[src_v], rows_v, sem).wait()

        def grp(t, c2):
            r = t // LG
            col = (t % LG) * 16
            v = rows_v[r, pl.ds(col, 16)] + ew_v[r, pl.ds(col, 16)]
            rows_v[r, pl.ds(col, 16)] = jnp.maximum(v, 0.0)
            return c2

        lax.fori_loop(0, CHUNK * LG, grp, 0, unroll=8)
        pltpu.sync_copy(rows_v, acc_sh.at[dst_v], add=True)
        return carry

    lax.fori_loop(0, NCHUNK, chunk, 0)

    plsc.subcore_barrier()
    obase = cid * N_NODES + rbase

    def crow(i, carry):
        r = pl.multiple_of(rbase + i * REM, 8)
        o = pl.multiple_of(obase + i * REM, 8)
        pltpu.sync_copy(acc_sh.at[pl.ds(r, REM)], zbuf_v)
        pltpu.sync_copy(zbuf_v, out_hbm.at[pl.ds(o, REM)])
        return carry

    lax.fori_loop(0, RPT // REM, crow, 0)

    @pl.when(sid == 0)
    def _copy_tail():
        pltpu.sync_copy(acc_sh.at[pl.ds(NS * RPT, REM)], zbuf_v)
        pltpu.sync_copy(zbuf_v,
                        out_hbm.at[pl.ds(cid * N_NODES + NS * RPT, REM)])


_edge_pass = functools.partial(
    pl.kernel,
    _edge_pass_body,
    out_type=jax.ShapeDtypeStruct((NC * N_NODES, H), jnp.float32),
    mesh=plsc.VectorSubcoreMesh(core_axis_name="c", subcore_axis_name="s"),
    scratch_types=[
        pltpu.VMEM((CHUNK,), jnp.int32),
        pltpu.VMEM((CHUNK,), jnp.int32),
        pltpu.VMEM((CHUNK, H), jnp.float32),
        pltpu.VMEM((CHUNK, H), jnp.float32),
        pltpu.VMEM((REM, H), jnp.float32),
        pltpu.VMEM_SHARED((N_NODES, H), jnp.float32),
        pltpu.SemaphoreType.DMA,
    ],
)()


def _update_body(h_ref, p_ref, Wt_ref, Wb_ref, bu_ref, Wn_ref, h_out, hw_out):
    agg = p_ref[0] + p_ref[1]
    hn = jnp.maximum(
        jnp.dot(h_ref[...], Wt_ref[...], preferred_element_type=jnp.float32)
        + jnp.dot(agg, Wb_ref[...], preferred_element_type=jnp.float32)
        + bu_ref[...][None, :], 0.0)
    h_out[...] = hn
    hw_out[...] = jnp.dot(hn, Wn_ref[...], preferred_element_type=jnp.float32)


def _update(h, aggp, Wt, Wb, bu, Wnext):
    return pl.pallas_call(
        _update_body,
        grid=(N_NODES // BLK_N,),
        in_specs=[
            pl.BlockSpec((BLK_N, H), lambda i: (i, 0)),
            pl.BlockSpec((NC, BLK_N, H), lambda i: (0, i, 0)),
            pl.BlockSpec((H, H), lambda i: (0, 0)),
            pl.BlockSpec((H, H), lambda i: (0, 0)),
            pl.BlockSpec((H,), lambda i: (0,)),
            pl.BlockSpec((H, H), lambda i: (0, 0)),
        ],
        out_specs=[
            pl.BlockSpec((BLK_N, H), lambda i: (i, 0)),
            pl.BlockSpec((BLK_N, H), lambda i: (i, 0)),
        ],
        out_shape=[jax.ShapeDtypeStruct((N_NODES, H), jnp.float32)] * 2,
    )(h, aggp, Wt, Wb, bu, Wnext)


def _update_final_body(h_ref, p_ref, Wt_ref, Wb_ref, bu_ref, xb_ref,
                       h_out, pool_out):
    i = pl.program_id(0)
    agg = p_ref[0] + p_ref[1]
    hn = jnp.maximum(
        jnp.dot(h_ref[...], Wt_ref[...], preferred_element_type=jnp.float32)
        + jnp.dot(agg, Wb_ref[...], preferred_element_type=jnp.float32)
        + bu_ref[...][None, :], 0.0)
    h_out[...] = hn
    ids = xb_ref[0, 0, :]
    oh = (lax.broadcasted_iota(jnp.int32, (B, BLK_N), 0)
          == ids[None, :]).astype(jnp.float32)
    contrib = jnp.dot(oh, hn, preferred_element_type=jnp.float32)

    @pl.when(i == 0)
    def _():
        pool_out[...] = jnp.zeros_like(pool_out)

    pool_out[...] += contrib


def _update_final(h, aggp, Wt, Wb, bu, xb3):
    return pl.pallas_call(
        _update_final_body,
        grid=(N_NODES // BLK_N,),
        in_specs=[
            pl.BlockSpec((BLK_N, H), lambda i: (i, 0)),
            pl.BlockSpec((NC, BLK_N, H), lambda i: (0, i, 0)),
            pl.BlockSpec((H, H), lambda i: (0, 0)),
            pl.BlockSpec((H, H), lambda i: (0, 0)),
            pl.BlockSpec((H,), lambda i: (0,)),
            pl.BlockSpec((1, 1, BLK_N), lambda i: (i, 0, 0)),
        ],
        out_specs=[
            pl.BlockSpec((BLK_N, H), lambda i: (i, 0)),
            pl.BlockSpec((B, H), lambda i: (0, 0)),
        ],
        out_shape=[
            jax.ShapeDtypeStruct((N_NODES, H), jnp.float32),
            jax.ShapeDtypeStruct((B, H), jnp.float32),
        ],
    )(h, aggp, Wt, Wb, bu, xb3)


def kernel(x_n, edge_index_n, edge_attr_n, x_n_batch,
           W_node, b_node, W_edge, b_edge, W_msg, b_msg, W_upd, b_upd):
    src = edge_index_n[0]
    dst = edge_index_n[1]
    Wm_top = W_msg[:, :H, :]
    Wm_bot = W_msg[:, H:, :]
    Wu_top = W_upd[:, :H, :]
    Wu_bot = W_upd[:, H:, :]

    h, hw = _node_encode(x_n, W_node, b_node, Wm_top[0])
    ews = _edge_precompute(edge_attr_n, W_edge, b_edge, Wm_bot, b_msg)
    zeros = jnp.zeros((REM, H), jnp.float32)
    xb3 = x_n_batch.reshape(N_NODES // BLK_N, 1, BLK_N)

    pooled = None
    for l in range(L):
        aggp = _edge_pass(hw, ews[l], src, dst, zeros)
        aggp = aggp.reshape(NC, N_NODES, H)
        if l < L - 1:
            h, hw = _update(h, aggp, Wu_top[l], Wu_bot[l], b_upd[l],
                            Wm_top[l + 1])
        else:
            h, pooled = _update_final(h, aggp, Wu_top[l], Wu_bot[l],
                                      b_upd[l], xb3)
    return h, pooled
